# baseline (speedup 1.0000x reference)
import jax
import jax.numpy as jnp
from jax.experimental import pallas as pl
from jax.experimental.pallas import tpu as pltpu

_N = 2048
_BW = 64
_W = 2048


def _rot_cols(x, s):
    if s == 0:
        return x
    return pltpu.roll(x, (-s) % _W, axis=1)


def _shift_up(x, nxt):
    return jnp.concatenate([x[1:], nxt], axis=0)


def _shift_dn(x):
    return jnp.concatenate([x[:1], x[:-1]], axis=0)


def _dc_block(a_ref, p_ref, n_ref, an_ref, pn_ref, nn_ref, pos_ref, neg_ref):
    b = pl.program_id(0)
    wrow = jax.lax.broadcasted_iota(jnp.int32, (_BW, 1), 0) + b * _BW
    last_w = wrow == (_N // 2 - 1)
    col = jax.lax.broadcasted_iota(jnp.int32, (1, _W), 1)
    odd_c = (col % 2) == 1
    c2046 = col == _N - 2
    c2047 = col == _N - 1

    ae = a_ref[:, 0, :]
    ao = a_ref[:, 1, :]
    aen = an_ref[0]

    def pool_pair(ce, co, cen):
        de = jnp.abs(ae - ce)
        do = jnp.abs(ao - co)
        den = jnp.abs(aen - cen)
        rd = (de, do, _shift_up(de, den))
        rc = (ce, co, _shift_up(ce, cen))

        bmaxd = bmind = bmaxc = bminc = None
        for dr in range(3):
            for dc in range(3):
                t = _rot_cols(rd[dr], dc)
                ct = _rot_cols(rc[dr], dc)
                if dr == 0 and dc == 0:
                    bmaxd = bmind = t
                    bmaxc = bminc = ct
                else:
                    mx = t > bmaxd
                    bmaxd = jnp.where(mx, t, bmaxd)
                    bmaxc = jnp.where(mx, ct, bmaxc)
                    mn = t < bmind
                    bmind = jnp.where(mn, t, bmind)
                    bminc = jnp.where(mn, ct, bminc)

        s = bminc + bmaxc
        f = jnp.where(odd_c, _rot_cols(s, -1), s)
        f = jnp.where(c2046, _rot_cols(s, -2), f)
        p0 = jnp.where(last_w, _shift_dn(f), f)
        p0 = jnp.where(c2047, ce + ce, p0)
        p1 = jnp.where(jnp.logical_or(last_w, c2047), co + co, f)
        return p0, p1

    p0, p1 = pool_pair(p_ref[:, 0, :], p_ref[:, 1, :], pn_ref[0])
    pos_ref[:, 0, :] = p0
    pos_ref[:, 1, :] = p1
    n0, n1 = pool_pair(n_ref[:, 0, :], n_ref[:, 1, :], nn_ref[0])
    neg_ref[:, 0, :] = n0
    neg_ref[:, 1, :] = n1


def _next_even_rows(x3):
    nb = (_N // 2) // _BW
    nxt = jnp.concatenate([x3[_BW::_BW, 0, :], x3[-1:, 0, :]], axis=0)
    return nxt.reshape(nb, 1, _W)


@jax.jit
def kernel(anchor, positive, negative):
    nh = _N // 2
    nb = nh // _BW
    a3 = anchor.reshape(nh, 2, _W)
    p3 = positive.reshape(nh, 2, _W)
    n3 = negative.reshape(nh, 2, _W)
    an, pn, nn = _next_even_rows(a3), _next_even_rows(p3), _next_even_rows(n3)
    blk = pl.BlockSpec((_BW, 2, _W), lambda b: (b, 0, 0))
    skinny = pl.BlockSpec((1, 1, _W), lambda b: (b, 0, 0))
    pos, neg = pl.pallas_call(
        _dc_block,
        grid=(nb,),
        in_specs=[blk, blk, blk, skinny, skinny, skinny],
        out_specs=[blk, blk],
        out_shape=[jax.ShapeDtypeStruct((nh, 2, _W), jnp.float32)] * 2,
    )(a3, p3, n3, an, pn, nn)
    return (pos.reshape(_N, _N), neg.reshape(_N, _N))


# speedup vs baseline: 1.0367x; 1.0367x over previous
import jax
import jax.numpy as jnp
from jax.experimental import pallas as pl
from jax.experimental.pallas import tpu as pltpu

_N = 2048
_BW = 32
_W = 2048


def _rot_cols(x, s):
    if s == 0:
        return x
    return jnp.concatenate([x[:, s:], x[:, :s]], axis=1)


def _shift_up(x, nxt):
    return jnp.concatenate([x[1:], nxt], axis=0)


def _shift_dn(x):
    return jnp.concatenate([x[:1], x[:-1]], axis=0)


def _dc_block(a_ref, p_ref, n_ref, an_ref, pn_ref, nn_ref, pos_ref, neg_ref):
    b = pl.program_id(0)
    wrow = jax.lax.broadcasted_iota(jnp.int32, (_BW, 1), 0) + b * _BW
    last_w = wrow == (_N // 2 - 1)
    col = jax.lax.broadcasted_iota(jnp.int32, (1, _W), 1)
    odd_c = (col % 2) == 1
    c2046 = col == _N - 2
    c2047 = col == _N - 1

    ae = a_ref[:, 0, :]
    ao = a_ref[:, 1, :]
    aen = an_ref[0]

    def pool_pair(ce, co, cen):
        de = jnp.abs(ae - ce)
        do = jnp.abs(ao - co)
        den = jnp.abs(aen - cen)
        rd = (de, do, _shift_up(de, den))
        rc = (ce, co, _shift_up(ce, cen))

        bmaxd = bmind = bmaxc = bminc = None
        for dr in range(3):
            for dc in range(3):
                t = _rot_cols(rd[dr], dc)
                ct = _rot_cols(rc[dr], dc)
                if dr == 0 and dc == 0:
                    bmaxd = bmind = t
                    bmaxc = bminc = ct
                else:
                    mx = t > bmaxd
                    bmaxd = jnp.where(mx, t, bmaxd)
                    bmaxc = jnp.where(mx, ct, bmaxc)
                    mn = t < bmind
                    bmind = jnp.where(mn, t, bmind)
                    bminc = jnp.where(mn, ct, bminc)

        s = bminc + bmaxc
        f = jnp.where(odd_c, _rot_cols(s, -1), s)
        f = jnp.where(c2046, _rot_cols(s, -2), f)
        p0 = jnp.where(last_w, _shift_dn(f), f)
        p0 = jnp.where(c2047, ce + ce, p0)
        p1 = jnp.where(jnp.logical_or(last_w, c2047), co + co, f)
        return p0, p1

    p0, p1 = pool_pair(p_ref[:, 0, :], p_ref[:, 1, :], pn_ref[0])
    pos_ref[:, 0, :] = p0
    pos_ref[:, 1, :] = p1
    n0, n1 = pool_pair(n_ref[:, 0, :], n_ref[:, 1, :], nn_ref[0])
    neg_ref[:, 0, :] = n0
    neg_ref[:, 1, :] = n1


def _next_even_rows(x3):
    nb = (_N // 2) // _BW
    nxt = jnp.concatenate([x3[_BW::_BW, 0, :], x3[-1:, 0, :]], axis=0)
    return nxt.reshape(nb, 1, _W)


@jax.jit
def kernel(anchor, positive, negative):
    nh = _N // 2
    nb = nh // _BW
    a3 = anchor.reshape(nh, 2, _W)
    p3 = positive.reshape(nh, 2, _W)
    n3 = negative.reshape(nh, 2, _W)
    an, pn, nn = _next_even_rows(a3), _next_even_rows(p3), _next_even_rows(n3)
    blk = pl.BlockSpec((_BW, 2, _W), lambda b: (b, 0, 0))
    skinny = pl.BlockSpec((1, 1, _W), lambda b: (b, 0, 0))
    pos, neg = pl.pallas_call(
        _dc_block,
        grid=(nb,),
        in_specs=[blk, blk, blk, skinny, skinny, skinny],
        out_specs=[blk, blk],
        out_shape=[jax.ShapeDtypeStruct((nh, 2, _W), jnp.float32)] * 2,
    )(a3, p3, n3, an, pn, nn)
    return (pos.reshape(_N, _N), neg.reshape(_N, _N))


# speedup vs baseline: 1.0885x; 1.0500x over previous
import jax
import jax.numpy as jnp
from jax.experimental import pallas as pl
from jax.experimental.pallas import tpu as pltpu

_N = 2048
_BW = 64
_W = 2048


def _rot_cols(x, s):
    if s == 0:
        return x
    return jnp.concatenate([x[:, s:], x[:, :s]], axis=1)


def _shift_up(x, nxt):
    return jnp.concatenate([x[1:], nxt], axis=0)


def _shift_dn(x):
    return jnp.concatenate([x[:1], x[:-1]], axis=0)


def _dc_block(a_ref, p_ref, n_ref, an_ref, pn_ref, nn_ref, pos_ref, neg_ref):
    b = pl.program_id(0)
    wrow = jax.lax.broadcasted_iota(jnp.int32, (_BW, 1), 0) + b * _BW
    last_w = wrow == (_N // 2 - 1)
    col = jax.lax.broadcasted_iota(jnp.int32, (1, _W), 1)
    odd_c = (col % 2) == 1
    c2046 = col == _N - 2
    c2047 = col == _N - 1

    ae = a_ref[:, 0, :]
    ao = a_ref[:, 1, :]
    aen = an_ref[0, :1, :]

    def pool_pair(ce, co, cen):
        de = jnp.abs(ae - ce)
        do = jnp.abs(ao - co)
        den = jnp.abs(aen - cen)
        rd = (de, do, _shift_up(de, den))
        rc = (ce, co, _shift_up(ce, cen))

        bmaxd = bmind = bmaxc = bminc = None
        for dr in range(3):
            for dc in range(3):
                t = _rot_cols(rd[dr], dc)
                ct = _rot_cols(rc[dr], dc)
                if dr == 0 and dc == 0:
                    bmaxd = bmind = t
                    bmaxc = bminc = ct
                else:
                    mx = t > bmaxd
                    bmaxd = jnp.where(mx, t, bmaxd)
                    bmaxc = jnp.where(mx, ct, bmaxc)
                    mn = t < bmind
                    bmind = jnp.where(mn, t, bmind)
                    bminc = jnp.where(mn, ct, bminc)

        s = bminc + bmaxc
        f = jnp.where(odd_c, _rot_cols(s, -1), s)
        f = jnp.where(c2046, _rot_cols(s, -2), f)
        p0 = jnp.where(last_w, _shift_dn(f), f)
        p0 = jnp.where(c2047, ce + ce, p0)
        p1 = jnp.where(jnp.logical_or(last_w, c2047), co + co, f)
        return p0, p1

    p0, p1 = pool_pair(p_ref[:, 0, :], p_ref[:, 1, :], pn_ref[0, :1, :])
    pos_ref[:, 0, :] = p0
    pos_ref[:, 1, :] = p1
    n0, n1 = pool_pair(n_ref[:, 0, :], n_ref[:, 1, :], nn_ref[0, :1, :])
    neg_ref[:, 0, :] = n0
    neg_ref[:, 1, :] = n1


@jax.jit
def kernel(anchor, positive, negative):
    nh = _N // 2
    nb = nh // _BW
    a3 = anchor.reshape(nh, 2, _W)
    p3 = positive.reshape(nh, 2, _W)
    n3 = negative.reshape(nh, 2, _W)
    blk = pl.BlockSpec((_BW, 2, _W), lambda b: (b, 0, 0))
    halo = pl.BlockSpec(
        (1, 2, _W), lambda b: (jnp.minimum((b + 1) * _BW, nh - 1), 0, 0))
    pos, neg = pl.pallas_call(
        _dc_block,
        grid=(nb,),
        in_specs=[blk, blk, blk, halo, halo, halo],
        out_specs=[blk, blk],
        out_shape=[jax.ShapeDtypeStruct((nh, 2, _W), jnp.float32)] * 2,
    )(a3, p3, n3, a3, p3, n3)
    return (pos.reshape(_N, _N), neg.reshape(_N, _N))
